# baseline (speedup 1.0000x reference)
import functools

import numpy as np
import jax
import jax.numpy as jnp
from jax import lax
from jax.experimental import pallas as pl
from jax.experimental.pallas import tpu as pltpu
from jax.experimental.pallas import tpu_sc as plsc

N = 62
NP = 64
NTRIL = N * (N + 1) // 2
ZSLOT = NTRIL
EW_PAD = 1968
ROWS_PER_TILE = 2


def _dense_to_tril_index() -> np.ndarray:
    i, j = np.meshgrid(np.arange(N), np.arange(NP), indexing="ij")
    a = np.maximum(i, j)
    b = np.minimum(i, j)
    t = a * (a + 1) // 2 + b
    t[:, N:] = ZSLOT
    return t.astype(np.int32)


_GIDX = _dense_to_tril_index()


def _sc_build_wd(ew_hbm, gidx_hbm, wd_hbm, ew_v, gidx_v, wd_v, sem1, sem2):
    wid = lax.axis_index("s") * 2 + lax.axis_index("c")

    @pl.when(wid == 0)
    def _():
        ew_v[pl.ds(EW_PAD - 16, 16)] = jnp.zeros((16,), jnp.float32)
        cp1 = pltpu.async_copy(ew_hbm, ew_v.at[pl.ds(0, NTRIL)], sem1)
        cp2 = pltpu.async_copy(gidx_hbm, gidx_v, sem2)
        cp1.wait()
        cp2.wait()
        for r in range(N):
            for c in range(NP // 16):
                idx = gidx_v[r, pl.ds(c * 16, 16)]
                wd_v[r, pl.ds(c * 16, 16)] = plsc.load_gather(ew_v, [idx])
        pltpu.sync_copy(wd_v, wd_hbm)


@functools.cache
def _build_wd():
    return pl.kernel(
        _sc_build_wd,
        out_type=jax.ShapeDtypeStruct((N, NP), jnp.float32),
        mesh=plsc.VectorSubcoreMesh(core_axis_name="c", subcore_axis_name="s"),
        scratch_types=[
            pltpu.VMEM((EW_PAD,), jnp.float32),
            pltpu.VMEM((N, NP), jnp.int32),
            pltpu.VMEM((N, NP), jnp.float32),
            pltpu.SemaphoreType.DMA,
            pltpu.SemaphoreType.DMA,
        ],
        compiler_params=pltpu.CompilerParams(needs_layout_passes=False),
    )


def _tc_body(wd_ref, x_ref, c_ref, linw_ref, linb_ref, c2b_ref, fcw_ref,
             fcb_ref, out_ref):
    Wd = wd_ref[...]
    absW = jnp.abs(Wd)
    deg_c = jnp.sum(absW, axis=1, keepdims=True)
    deg_r = jnp.sum(absW, axis=0, keepdims=True)
    dis_c = jnp.where(deg_c > 0,
                      lax.rsqrt(jnp.where(deg_c > 0, deg_c, 1.0)), 0.0)
    dis_r = jnp.where(deg_r > 0,
                      lax.rsqrt(jnp.where(deg_r > 0, deg_r, 1.0)), 0.0)
    A = Wd * dis_c * dis_r
    cv = c_ref[...]
    u_row = jnp.sum(A * cv, axis=0, keepdims=True)
    w = jnp.sum(A * u_row, axis=1, keepdims=True)
    X = x_ref[...]
    V = jnp.sum(X * w[None, :, :], axis=1)
    bias = jnp.sum(cv) * linb_ref[...] + c2b_ref[0, 0]
    Y = lax.dot_general(V, linw_ref[...], (((1,), (1,)), ((), ())),
                        preferred_element_type=jnp.float32,
                        precision=lax.Precision.HIGHEST) + bias
    Y = jnp.maximum(Y, 0.0)
    out_ref[...] = lax.dot_general(Y, fcw_ref[...], (((1,), (1,)), ((), ())),
                                   preferred_element_type=jnp.float32,
                                   precision=lax.Precision.HIGHEST) \
        + fcb_ref[...]


def _tc_call(wd, X, cvec, lin_W, lin_b2, c2b, fc_W, fc_b2):
    return pl.pallas_call(
        _tc_body,
        out_shape=jax.ShapeDtypeStruct((X.shape[0], fc_W.shape[0]),
                                       jnp.float32),
    )(wd, X, cvec, lin_W, lin_b2, c2b, fc_W, fc_b2)


def kernel(X, ew, lin_W, lin_b, conv2_w, conv2_b, fc_W, fc_b, edge_index):
    del edge_index
    wd = _build_wd()(ew, jnp.asarray(_GIDX))
    cvec = conv2_w.reshape(N, 1)
    out = _tc_call(wd, X, cvec, lin_W, lin_b.reshape(1, -1),
                   conv2_b.reshape(1, 1), fc_W, fc_b.reshape(1, -1))
    return out


# speedup vs baseline: 1.0504x; 1.0504x over previous
import functools

import numpy as np
import jax
import jax.numpy as jnp
from jax import lax
from jax.experimental import pallas as pl
from jax.experimental.pallas import tpu as pltpu
from jax.experimental.pallas import tpu_sc as plsc

N = 62
NP = 64
NTRIL = N * (N + 1) // 2
ZSLOT = NTRIL
EW_PAD = 1968
ROWS_PER_TILE = 2


def _dense_to_tril_index() -> np.ndarray:
    i, j = np.meshgrid(np.arange(N), np.arange(NP), indexing="ij")
    a = np.maximum(i, j)
    b = np.minimum(i, j)
    t = a * (a + 1) // 2 + b
    t[:, N:] = ZSLOT
    return t.astype(np.int32)


_GIDX = _dense_to_tril_index()


def _sc_build_wd(ew_hbm, gidx_hbm, wd_hbm, ew_v, gidx_v, wd_v, sem1, sem2):
    wid = lax.axis_index("s") + lax.axis_index("c")

    @pl.when(wid == 0)
    def _():
        ew_v[pl.ds(EW_PAD - 16, 16)] = jnp.zeros((16,), jnp.float32)
        cp1 = pltpu.async_copy(ew_hbm, ew_v.at[pl.ds(0, NTRIL)], sem1)
        cp2 = pltpu.async_copy(gidx_hbm, gidx_v, sem2)
        cp1.wait()
        cp2.wait()
        for r in range(N):
            for c in range(NP // 16):
                idx = gidx_v[r, pl.ds(c * 16, 16)]
                wd_v[r, pl.ds(c * 16, 16)] = plsc.load_gather(ew_v, [idx])
        pltpu.sync_copy(wd_v, wd_hbm)


@functools.cache
def _build_wd():
    return pl.kernel(
        _sc_build_wd,
        out_type=jax.ShapeDtypeStruct((N, NP), jnp.float32),
        mesh=plsc.VectorSubcoreMesh(core_axis_name="c", subcore_axis_name="s",
                                    num_cores=1),
        scratch_types=[
            pltpu.VMEM((EW_PAD,), jnp.float32),
            pltpu.VMEM((N, NP), jnp.int32),
            pltpu.VMEM((N, NP), jnp.float32),
            pltpu.SemaphoreType.DMA,
            pltpu.SemaphoreType.DMA,
        ],
        compiler_params=pltpu.CompilerParams(needs_layout_passes=False),
    )


def _tc_body(wd_ref, x_ref, c_ref, linw_ref, linb_ref, c2b_ref, fcw_ref,
             fcb_ref, out_ref):
    Wd = wd_ref[...]
    absW = jnp.abs(Wd)
    deg_c = jnp.sum(absW, axis=1, keepdims=True)
    deg_r = jnp.sum(absW, axis=0, keepdims=True)
    dis_c = jnp.where(deg_c > 0,
                      lax.rsqrt(jnp.where(deg_c > 0, deg_c, 1.0)), 0.0)
    dis_r = jnp.where(deg_r > 0,
                      lax.rsqrt(jnp.where(deg_r > 0, deg_r, 1.0)), 0.0)
    A = Wd * dis_c * dis_r
    cv = c_ref[...]
    u_row = jnp.sum(A * cv, axis=0, keepdims=True)
    w = jnp.sum(A * u_row, axis=1, keepdims=True)
    X = x_ref[...]
    V = jnp.sum(X * w[None, :, :], axis=1)
    bias = jnp.sum(cv) * linb_ref[...] + c2b_ref[0, 0]
    Y = lax.dot_general(V, linw_ref[...], (((1,), (1,)), ((), ())),
                        preferred_element_type=jnp.float32,
                        precision=lax.Precision.HIGHEST) + bias
    Y = jnp.maximum(Y, 0.0)
    out_ref[...] = lax.dot_general(Y, fcw_ref[...], (((1,), (1,)), ((), ())),
                                   preferred_element_type=jnp.float32,
                                   precision=lax.Precision.HIGHEST) \
        + fcb_ref[...]


def _tc_call(wd, X, cvec, lin_W, lin_b2, c2b, fc_W, fc_b2):
    return pl.pallas_call(
        _tc_body,
        out_shape=jax.ShapeDtypeStruct((X.shape[0], fc_W.shape[0]),
                                       jnp.float32),
    )(wd, X, cvec, lin_W, lin_b2, c2b, fc_W, fc_b2)


def kernel(X, ew, lin_W, lin_b, conv2_w, conv2_b, fc_W, fc_b, edge_index):
    del edge_index
    wd = _build_wd()(ew, jnp.asarray(_GIDX))
    cvec = conv2_w.reshape(N, 1)
    out = _tc_call(wd, X, cvec, lin_W, lin_b.reshape(1, -1),
                   conv2_b.reshape(1, 1), fc_W, fc_b.reshape(1, -1))
    return out


# speedup vs baseline: 1.0602x; 1.0093x over previous
import functools

import numpy as np
import jax
import jax.numpy as jnp
from jax import lax
from jax.experimental import pallas as pl
from jax.experimental.pallas import tpu as pltpu
from jax.experimental.pallas import tpu_sc as plsc

N = 62
NP = 64
NTRIL = N * (N + 1) // 2
ZSLOT = NTRIL
EW_PAD = 1968
ROWS_PER_TILE = 2


def _dense_to_tril_index() -> np.ndarray:
    i, j = np.meshgrid(np.arange(N), np.arange(NP), indexing="ij")
    a = np.maximum(i, j)
    b = np.minimum(i, j)
    t = a * (a + 1) // 2 + b
    t[:, N:] = ZSLOT
    return t.astype(np.int32)


_GIDX = _dense_to_tril_index()


def _sc_build_wd(ew_hbm, wd_hbm, ew_v, wd_v):
    wid = lax.axis_index("s") + lax.axis_index("c")

    @pl.when(wid == 0)
    def _():
        ew_v[pl.ds(EW_PAD - 16, 16)] = jnp.zeros((16,), jnp.float32)
        pltpu.sync_copy(ew_hbm, ew_v.at[pl.ds(0, NTRIL)])
        lanes = lax.iota(jnp.int32, 16)
        for c in range(NP // 16):
            j = lanes + (16 * c)
            in_bounds = j < N
            for r in range(N):
                a = jnp.maximum(j, r)
                b = jnp.minimum(j, r)
                t = lax.shift_right_logical(a * (a + 1), 1) + b
                idx = jnp.where(in_bounds, t, ZSLOT)
                wd_v[r, pl.ds(c * 16, 16)] = plsc.load_gather(ew_v, [idx])
        pltpu.sync_copy(wd_v, wd_hbm)


@functools.cache
def _build_wd():
    return pl.kernel(
        _sc_build_wd,
        out_type=jax.ShapeDtypeStruct((N, NP), jnp.float32),
        mesh=plsc.VectorSubcoreMesh(core_axis_name="c", subcore_axis_name="s",
                                    num_cores=1),
        scratch_types=[
            pltpu.VMEM((EW_PAD,), jnp.float32),
            pltpu.VMEM((N, NP), jnp.float32),
        ],
        compiler_params=pltpu.CompilerParams(needs_layout_passes=False),
    )


def _tc_body(wd_ref, x_ref, c_ref, linw_ref, linb_ref, c2b_ref, fcw_ref,
             fcb_ref, out_ref):
    Wd = wd_ref[...]
    absW = jnp.abs(Wd)
    deg_c = jnp.sum(absW, axis=1, keepdims=True)
    deg_r = jnp.sum(absW, axis=0, keepdims=True)
    dis_c = jnp.where(deg_c > 0,
                      lax.rsqrt(jnp.where(deg_c > 0, deg_c, 1.0)), 0.0)
    dis_r = jnp.where(deg_r > 0,
                      lax.rsqrt(jnp.where(deg_r > 0, deg_r, 1.0)), 0.0)
    A = Wd * dis_c * dis_r
    cv = c_ref[...]
    u_row = jnp.sum(A * cv, axis=0, keepdims=True)
    w = jnp.sum(A * u_row, axis=1, keepdims=True)
    X = x_ref[...]
    V = jnp.sum(X * w[None, :, :], axis=1)
    bias = jnp.sum(cv) * linb_ref[...] + c2b_ref[0, 0]
    Y = lax.dot_general(V, linw_ref[...], (((1,), (1,)), ((), ())),
                        preferred_element_type=jnp.float32,
                        precision=lax.Precision.HIGHEST) + bias
    Y = jnp.maximum(Y, 0.0)
    out_ref[...] = lax.dot_general(Y, fcw_ref[...], (((1,), (1,)), ((), ())),
                                   preferred_element_type=jnp.float32,
                                   precision=lax.Precision.HIGHEST) \
        + fcb_ref[...]


def _tc_call(wd, X, cvec, lin_W, lin_b2, c2b, fc_W, fc_b2):
    return pl.pallas_call(
        _tc_body,
        out_shape=jax.ShapeDtypeStruct((X.shape[0], fc_W.shape[0]),
                                       jnp.float32),
    )(wd, X, cvec, lin_W, lin_b2, c2b, fc_W, fc_b2)


def kernel(X, ew, lin_W, lin_b, conv2_w, conv2_b, fc_W, fc_b, edge_index):
    del edge_index
    wd = _build_wd()(ew)
    cvec = conv2_w.reshape(N, 1)
    out = _tc_call(wd, X, cvec, lin_W, lin_b.reshape(1, -1),
                   conv2_b.reshape(1, 1), fc_W, fc_b.reshape(1, -1))
    return out
